# baseline (speedup 1.0000x reference)
import functools

import jax
import jax.numpy as jnp
from jax import lax
from jax.experimental import pallas as pl
from jax.experimental.pallas import tpu as pltpu
from jax.experimental.pallas import tpu_sc as plsc

N_NODES = 10000
D = 128
N_EDGES = 320000
NC, NS, L = 2, 16, 16
NW = NC * NS
CHUNK = 128
N_CHUNKS = 80
E_PER_TILE = N_CHUNKS * CHUNK
PAD_E = NW * E_PER_TILE
G = 16
NG = N_CHUNKS // G
PAD_NODES = 10112
ROWS_PER_TILE = PAD_NODES // NS

_BLK = 1000
_GRID = N_NODES // _BLK


def _h_body(f_ref, n_ref, o_ref):
    o_ref[...] = f_ref[...] * n_ref[...]


def _final_body(p_ref, n_ref, w_ref, b_ref, o_ref):
    agg = (p_ref[0] + p_ref[1]) * n_ref[...]
    o_ref[...] = lax.dot_general(
        agg, w_ref[...], (((1,), (1,)), ((), ())),
        preferred_element_type=jnp.float32) + b_ref[...]


def _sc_body(src_hbm, dst_hbm, h_hbm, out_hbm,
             sidx, didx, rows0, rows1, agg_sh, sem0, sem1, semi):
    cid = lax.axis_index("c")
    sid = lax.axis_index("s")
    wid = sid * NC + cid
    stripe = sid * ROWS_PER_TILE

    def zfill(r, carry):
        for c in range(D // L):
            rows0[r, pl.ds(c * L, L)] = jnp.zeros((L,), jnp.float32)
        return carry

    lax.fori_loop(0, CHUNK, zfill, 0)

    def zcopy(i, carry):
        pltpu.sync_copy(rows0, agg_sh.at[pl.ds(stripe + i * CHUNK, CHUNK)])
        return carry

    nz = ROWS_PER_TILE // CHUNK
    lax.fori_loop(0, nz, zcopy, 0)
    rem = ROWS_PER_TILE - nz * CHUNK
    pltpu.sync_copy(rows0.at[pl.ds(0, rem)],
                    agg_sh.at[pl.ds(stripe + nz * CHUNK, rem)])

    pltpu.sync_copy(src_hbm.at[wid, pl.ds(0, G)], sidx.at[pl.ds(0, G)])
    pltpu.sync_copy(dst_hbm.at[wid, pl.ds(0, G)], didx.at[pl.ds(0, G)])
    plsc.subcore_barrier()
    pltpu.async_copy(h_hbm.at[sidx.at[0]], rows0, sem0)

    def group(g, carry):
        base = (g % 2) * G
        nbase = G - base
        gn = ((g + 1) % NG) * G
        pltpu.async_copy(src_hbm.at[wid, pl.ds(gn, G)],
                         sidx.at[pl.ds(nbase, G)], semi)
        pltpu.async_copy(dst_hbm.at[wid, pl.ds(gn, G)],
                         didx.at[pl.ds(nbase, G)], semi)
        for k in range(G):
            cur, csem = (rows0, sem0) if k % 2 == 0 else (rows1, sem1)
            nxt, nsem = (rows1, sem1) if k % 2 == 0 else (rows0, sem0)
            pltpu.make_async_copy(h_hbm.at[sidx.at[base + k]], cur, csem).wait()
            if k < G - 1:
                pltpu.async_copy(h_hbm.at[sidx.at[base + k + 1]], nxt, nsem)
            else:
                pltpu.make_async_copy(src_hbm.at[wid, pl.ds(gn, G)],
                                      sidx.at[pl.ds(nbase, G)], semi).wait()
                pltpu.make_async_copy(dst_hbm.at[wid, pl.ds(gn, G)],
                                      didx.at[pl.ds(nbase, G)], semi).wait()
                pltpu.async_copy(h_hbm.at[sidx.at[nbase]], nxt, nsem)
            pltpu.sync_copy(cur, agg_sh.at[didx.at[base + k]], add=True)
        return carry

    lax.fori_loop(0, NG, group, 0)
    pltpu.make_async_copy(h_hbm.at[sidx.at[0]], rows0, sem0).wait()
    plsc.subcore_barrier()

    pltpu.sync_copy(agg_sh.at[pl.ds(stripe, ROWS_PER_TILE)],
                    out_hbm.at[cid, pl.ds(stripe, ROWS_PER_TILE)])


_sc_agg = functools.partial(
    pl.kernel,
    out_type=jax.ShapeDtypeStruct((NC, PAD_NODES, D), jnp.float32),
    mesh=plsc.VectorSubcoreMesh(
        core_axis_name="c", subcore_axis_name="s",
        num_cores=NC, num_subcores=NS),
    scratch_types=[
        pltpu.VMEM((2 * G, CHUNK), jnp.int32),
        pltpu.VMEM((2 * G, CHUNK), jnp.int32),
        pltpu.VMEM((CHUNK, D), jnp.float32),
        pltpu.VMEM((CHUNK, D), jnp.float32),
        pltpu.VMEM_SHARED((PAD_NODES, D), jnp.float32),
        pltpu.SemaphoreType.DMA,
        pltpu.SemaphoreType.DMA,
        pltpu.SemaphoreType.DMA,
    ],
)(_sc_body)


def kernel(feature, norm, edge_index, W, b):
    src = edge_index[0].astype(jnp.int32)
    dst = edge_index[1].astype(jnp.int32)
    npad = PAD_E - N_EDGES
    pad_ids = jnp.arange(npad, dtype=jnp.int32)
    src = jnp.concatenate(
        [src, pad_ids % N_NODES]).reshape(NW, N_CHUNKS, CHUNK)
    dst = jnp.concatenate(
        [dst, N_NODES + pad_ids % (PAD_NODES - N_NODES)]
    ).reshape(NW, N_CHUNKS, CHUNK)

    h = pl.pallas_call(
        _h_body,
        grid=(_GRID,),
        in_specs=[
            pl.BlockSpec((_BLK, D), lambda i: (i, 0)),
            pl.BlockSpec((_BLK, 1), lambda i: (i, 0)),
        ],
        out_specs=pl.BlockSpec((_BLK, D), lambda i: (i, 0)),
        out_shape=jax.ShapeDtypeStruct((N_NODES, D), jnp.float32),
    )(feature, norm)

    partials = _sc_agg(src, dst, h)

    out = pl.pallas_call(
        _final_body,
        grid=(_GRID,),
        in_specs=[
            pl.BlockSpec((NC, _BLK, D), lambda i: (0, i, 0)),
            pl.BlockSpec((_BLK, 1), lambda i: (i, 0)),
            pl.BlockSpec((D, D), lambda i: (0, 0)),
            pl.BlockSpec((1, D), lambda i: (0, 0)),
        ],
        out_specs=pl.BlockSpec((_BLK, D), lambda i: (i, 0)),
        out_shape=jax.ShapeDtypeStruct((N_NODES, D), jnp.float32),
    )(partials, norm, W, b.reshape(1, D))

    return out


# speedup vs baseline: 1.0461x; 1.0461x over previous
import functools

import jax
import jax.numpy as jnp
from jax import lax
from jax.experimental import pallas as pl
from jax.experimental.pallas import tpu as pltpu
from jax.experimental.pallas import tpu_sc as plsc

N_NODES = 10000
D = 128
N_EDGES = 320000
NC, NS, L = 2, 16, 16
NW = NC * NS
CHUNK = 160
N_CHUNKS = 64
E_PER_TILE = N_CHUNKS * CHUNK
PAD_E = NW * E_PER_TILE
G = 8
NG = N_CHUNKS // G
PAD_NODES = 10112
ROWS_PER_TILE = PAD_NODES // NS

_BLK = 1000
_GRID = N_NODES // _BLK


def _h_body(f_ref, n_ref, o_ref):
    o_ref[...] = f_ref[...] * n_ref[...]


def _final_body(p_ref, n_ref, w_ref, b_ref, o_ref):
    agg = (p_ref[0] + p_ref[1]) * n_ref[...]
    o_ref[...] = lax.dot_general(
        agg, w_ref[...], (((1,), (1,)), ((), ())),
        preferred_element_type=jnp.float32) + b_ref[...]


def _sc_body(src_hbm, dst_hbm, h_hbm, out_hbm,
             sidx, didx, rows0, rows1, agg_sh, sem0, sem1, semi):
    cid = lax.axis_index("c")
    sid = lax.axis_index("s")
    wid = sid * NC + cid
    stripe = sid * ROWS_PER_TILE

    def zfill(r, carry):
        for c in range(D // L):
            rows0[r, pl.ds(c * L, L)] = jnp.zeros((L,), jnp.float32)
        return carry

    lax.fori_loop(0, CHUNK, zfill, 0)

    def zcopy(i, carry):
        pltpu.sync_copy(rows0, agg_sh.at[pl.ds(stripe + i * CHUNK, CHUNK)])
        return carry

    nz = ROWS_PER_TILE // CHUNK
    lax.fori_loop(0, nz, zcopy, 0)
    rem = ROWS_PER_TILE - nz * CHUNK
    pltpu.sync_copy(rows0.at[pl.ds(0, rem)],
                    agg_sh.at[pl.ds(stripe + nz * CHUNK, rem)])

    GE = G * CHUNK
    pltpu.sync_copy(src_hbm.at[wid, pl.ds(0, GE)], sidx.at[pl.ds(0, GE)])
    pltpu.sync_copy(dst_hbm.at[wid, pl.ds(0, GE)], didx.at[pl.ds(0, GE)])
    plsc.subcore_barrier()
    pltpu.async_copy(h_hbm.at[sidx.at[pl.ds(0, CHUNK)]], rows0, sem0)

    def group(g, carry):
        base = (g % 2) * GE
        nbase = GE - base
        gn = ((g + 1) % NG) * GE
        pltpu.async_copy(src_hbm.at[wid, pl.ds(gn, GE)],
                         sidx.at[pl.ds(nbase, GE)], semi)
        pltpu.async_copy(dst_hbm.at[wid, pl.ds(gn, GE)],
                         didx.at[pl.ds(nbase, GE)], semi)
        for k in range(G):
            cur, csem = (rows0, sem0) if k % 2 == 0 else (rows1, sem1)
            nxt, nsem = (rows1, sem1) if k % 2 == 0 else (rows0, sem0)
            pltpu.make_async_copy(
                h_hbm.at[sidx.at[pl.ds(base + k * CHUNK, CHUNK)]],
                cur, csem).wait()
            if k < G - 1:
                pltpu.async_copy(
                    h_hbm.at[sidx.at[pl.ds(base + (k + 1) * CHUNK, CHUNK)]],
                    nxt, nsem)
            else:
                pltpu.make_async_copy(src_hbm.at[wid, pl.ds(gn, GE)],
                                      sidx.at[pl.ds(nbase, GE)], semi).wait()
                pltpu.make_async_copy(dst_hbm.at[wid, pl.ds(gn, GE)],
                                      didx.at[pl.ds(nbase, GE)], semi).wait()
                pltpu.async_copy(h_hbm.at[sidx.at[pl.ds(nbase, CHUNK)]],
                                 nxt, nsem)
            pltpu.sync_copy(
                cur, agg_sh.at[didx.at[pl.ds(base + k * CHUNK, CHUNK)]],
                add=True)
        return carry

    lax.fori_loop(0, NG, group, 0)
    pltpu.make_async_copy(h_hbm.at[sidx.at[pl.ds(0, CHUNK)]],
                          rows0, sem0).wait()
    plsc.subcore_barrier()

    pltpu.sync_copy(agg_sh.at[pl.ds(stripe, ROWS_PER_TILE)],
                    out_hbm.at[cid, pl.ds(stripe, ROWS_PER_TILE)])


_sc_agg = functools.partial(
    pl.kernel,
    out_type=jax.ShapeDtypeStruct((NC, PAD_NODES, D), jnp.float32),
    mesh=plsc.VectorSubcoreMesh(
        core_axis_name="c", subcore_axis_name="s",
        num_cores=NC, num_subcores=NS),
    scratch_types=[
        pltpu.VMEM((2 * G * CHUNK,), jnp.int32),
        pltpu.VMEM((2 * G * CHUNK,), jnp.int32),
        pltpu.VMEM((CHUNK, D), jnp.float32),
        pltpu.VMEM((CHUNK, D), jnp.float32),
        pltpu.VMEM_SHARED((PAD_NODES, D), jnp.float32),
        pltpu.SemaphoreType.DMA,
        pltpu.SemaphoreType.DMA,
        pltpu.SemaphoreType.DMA,
    ],
)(_sc_body)


def kernel(feature, norm, edge_index, W, b):
    src = edge_index[0].astype(jnp.int32)
    dst = edge_index[1].astype(jnp.int32)
    npad = PAD_E - N_EDGES
    pad_ids = jnp.arange(npad, dtype=jnp.int32)
    src = jnp.concatenate(
        [src, pad_ids % N_NODES]).reshape(NW, N_CHUNKS * CHUNK)
    dst = jnp.concatenate(
        [dst, N_NODES + pad_ids % (PAD_NODES - N_NODES)]
    ).reshape(NW, N_CHUNKS * CHUNK)

    h = pl.pallas_call(
        _h_body,
        grid=(_GRID,),
        in_specs=[
            pl.BlockSpec((_BLK, D), lambda i: (i, 0)),
            pl.BlockSpec((_BLK, 1), lambda i: (i, 0)),
        ],
        out_specs=pl.BlockSpec((_BLK, D), lambda i: (i, 0)),
        out_shape=jax.ShapeDtypeStruct((N_NODES, D), jnp.float32),
    )(feature, norm)

    partials = _sc_agg(src, dst, h)

    out = pl.pallas_call(
        _final_body,
        grid=(_GRID,),
        in_specs=[
            pl.BlockSpec((NC, _BLK, D), lambda i: (0, i, 0)),
            pl.BlockSpec((_BLK, 1), lambda i: (i, 0)),
            pl.BlockSpec((D, D), lambda i: (0, 0)),
            pl.BlockSpec((1, D), lambda i: (0, 0)),
        ],
        out_specs=pl.BlockSpec((_BLK, D), lambda i: (i, 0)),
        out_shape=jax.ShapeDtypeStruct((N_NODES, D), jnp.float32),
    )(partials, norm, W, b.reshape(1, D))

    return out


# speedup vs baseline: 1.0527x; 1.0064x over previous
import functools

import jax
import jax.numpy as jnp
from jax import lax
from jax.experimental import pallas as pl
from jax.experimental.pallas import tpu as pltpu
from jax.experimental.pallas import tpu_sc as plsc

N_NODES = 10000
D = 128
N_EDGES = 320000
NC, NS, L = 2, 16, 16
NW = NC * NS
CHUNK = 160
N_CHUNKS = 64
E_PER_TILE = N_CHUNKS * CHUNK
PAD_E = NW * E_PER_TILE
G = 8
NG = N_CHUNKS // G
PAD_NODES = 10112
ROWS_PER_TILE = PAD_NODES // NS

_BLK = 1000
_GRID = N_NODES // _BLK


def _h_body(f_ref, n_ref, o_ref):
    o_ref[...] = f_ref[...] * n_ref[...]


def _final_body(p_ref, n_ref, w_ref, b_ref, o_ref):
    agg = (p_ref[0] + p_ref[1]) * n_ref[...]
    o_ref[...] = lax.dot_general(
        agg, w_ref[...], (((1,), (1,)), ((), ())),
        preferred_element_type=jnp.float32) + b_ref[...]


def _sc_body(src_hbm, dst_hbm, h_hbm, out_hbm,
             sidx, didx, rows0, rows1, agg_sh, sem0, sem1, semi):
    cid = lax.axis_index("c")
    sid = lax.axis_index("s")
    wid = sid * NC + cid
    stripe = sid * ROWS_PER_TILE

    GE = G * CHUNK
    pltpu.async_copy(src_hbm.at[wid, pl.ds(0, GE)],
                     sidx.at[pl.ds(0, GE)], semi)
    pltpu.async_copy(dst_hbm.at[wid, pl.ds(0, GE)],
                     didx.at[pl.ds(0, GE)], semi)

    def zfill(r, carry):
        for c in range(D // L):
            rows0[r, pl.ds(c * L, L)] = jnp.zeros((L,), jnp.float32)
        return carry

    lax.fori_loop(0, CHUNK, zfill, 0)

    def zcopy(i, carry):
        pltpu.sync_copy(rows0, agg_sh.at[pl.ds(stripe + i * CHUNK, CHUNK)])
        return carry

    nz = ROWS_PER_TILE // CHUNK
    lax.fori_loop(0, nz, zcopy, 0)
    rem = ROWS_PER_TILE - nz * CHUNK
    pltpu.sync_copy(rows0.at[pl.ds(0, rem)],
                    agg_sh.at[pl.ds(stripe + nz * CHUNK, rem)])

    pltpu.make_async_copy(src_hbm.at[wid, pl.ds(0, GE)],
                          sidx.at[pl.ds(0, GE)], semi).wait()
    pltpu.make_async_copy(dst_hbm.at[wid, pl.ds(0, GE)],
                          didx.at[pl.ds(0, GE)], semi).wait()
    plsc.subcore_barrier()
    pltpu.async_copy(h_hbm.at[sidx.at[pl.ds(0, CHUNK)]], rows0, sem0)

    def group(g, carry):
        base = (g % 2) * GE
        nbase = GE - base
        gn = ((g + 1) % NG) * GE
        pltpu.async_copy(src_hbm.at[wid, pl.ds(gn, GE)],
                         sidx.at[pl.ds(nbase, GE)], semi)
        pltpu.async_copy(dst_hbm.at[wid, pl.ds(gn, GE)],
                         didx.at[pl.ds(nbase, GE)], semi)
        for k in range(G):
            cur, csem = (rows0, sem0) if k % 2 == 0 else (rows1, sem1)
            nxt, nsem = (rows1, sem1) if k % 2 == 0 else (rows0, sem0)
            pltpu.make_async_copy(
                h_hbm.at[sidx.at[pl.ds(base + k * CHUNK, CHUNK)]],
                cur, csem).wait()
            if k < G - 1:
                pltpu.async_copy(
                    h_hbm.at[sidx.at[pl.ds(base + (k + 1) * CHUNK, CHUNK)]],
                    nxt, nsem)
            else:
                pltpu.make_async_copy(src_hbm.at[wid, pl.ds(gn, GE)],
                                      sidx.at[pl.ds(nbase, GE)], semi).wait()
                pltpu.make_async_copy(dst_hbm.at[wid, pl.ds(gn, GE)],
                                      didx.at[pl.ds(nbase, GE)], semi).wait()
                pltpu.async_copy(h_hbm.at[sidx.at[pl.ds(nbase, CHUNK)]],
                                 nxt, nsem)
            pltpu.sync_copy(
                cur, agg_sh.at[didx.at[pl.ds(base + k * CHUNK, CHUNK)]],
                add=True)
        return carry

    lax.fori_loop(0, NG, group, 0)
    pltpu.make_async_copy(h_hbm.at[sidx.at[pl.ds(0, CHUNK)]],
                          rows0, sem0).wait()
    plsc.subcore_barrier()

    pltpu.sync_copy(agg_sh.at[pl.ds(stripe, ROWS_PER_TILE)],
                    out_hbm.at[cid, pl.ds(stripe, ROWS_PER_TILE)])


_sc_agg = functools.partial(
    pl.kernel,
    out_type=jax.ShapeDtypeStruct((NC, PAD_NODES, D), jnp.float32),
    mesh=plsc.VectorSubcoreMesh(
        core_axis_name="c", subcore_axis_name="s",
        num_cores=NC, num_subcores=NS),
    scratch_types=[
        pltpu.VMEM((2 * G * CHUNK,), jnp.int32),
        pltpu.VMEM((2 * G * CHUNK,), jnp.int32),
        pltpu.VMEM((CHUNK, D), jnp.float32),
        pltpu.VMEM((CHUNK, D), jnp.float32),
        pltpu.VMEM_SHARED((PAD_NODES, D), jnp.float32),
        pltpu.SemaphoreType.DMA,
        pltpu.SemaphoreType.DMA,
        pltpu.SemaphoreType.DMA,
    ],
)(_sc_body)


def kernel(feature, norm, edge_index, W, b):
    src = edge_index[0].astype(jnp.int32)
    dst = edge_index[1].astype(jnp.int32)
    npad = PAD_E - N_EDGES
    pad_ids = jnp.arange(npad, dtype=jnp.int32)
    src = jnp.concatenate(
        [src, pad_ids % N_NODES]).reshape(NW, N_CHUNKS * CHUNK)
    dst = jnp.concatenate(
        [dst, N_NODES + pad_ids % (PAD_NODES - N_NODES)]
    ).reshape(NW, N_CHUNKS * CHUNK)

    h = pl.pallas_call(
        _h_body,
        grid=(_GRID,),
        in_specs=[
            pl.BlockSpec((_BLK, D), lambda i: (i, 0)),
            pl.BlockSpec((_BLK, 1), lambda i: (i, 0)),
        ],
        out_specs=pl.BlockSpec((_BLK, D), lambda i: (i, 0)),
        out_shape=jax.ShapeDtypeStruct((N_NODES, D), jnp.float32),
    )(feature, norm)

    partials = _sc_agg(src, dst, h)

    out = pl.pallas_call(
        _final_body,
        grid=(_GRID,),
        in_specs=[
            pl.BlockSpec((NC, _BLK, D), lambda i: (0, i, 0)),
            pl.BlockSpec((_BLK, 1), lambda i: (i, 0)),
            pl.BlockSpec((D, D), lambda i: (0, 0)),
            pl.BlockSpec((1, D), lambda i: (0, 0)),
        ],
        out_specs=pl.BlockSpec((_BLK, D), lambda i: (i, 0)),
        out_shape=jax.ShapeDtypeStruct((N_NODES, D), jnp.float32),
    )(partials, norm, W, b.reshape(1, D))

    return out
